# Initial kernel scaffold; baseline (speedup 1.0000x reference)
import jax
import jax.numpy as jnp
from jax.experimental import pallas as pl


def kernel(c, delta, idx):
    raise NotImplementedError("write your pallas kernel here")


# speedup vs baseline: 8.5492x; 8.5492x over previous
import jax
import jax.numpy as jnp
from jax.experimental import pallas as pl

_TARGET = 0
_TEST = 0.0

_ROWS = 32768
_COLS = 256
_BLOCK_ROWS = 2048


def _ifelse_kernel(c_ref, d_ref, oc_ref, od_ref):
    c = c_ref[...]
    d = d_ref[...]
    c0 = c[:, _TARGET:_TARGET + 1]
    d0 = d[:, _TARGET:_TARGET + 1]
    lo = c0 - d0
    hi = c0 + d0
    left = lo <= _TEST
    right = hi > _TEST
    min_hi = jnp.minimum(hi, _TEST)
    cl = (lo + min_hi) * 0.5
    dl = (min_hi - lo) * 0.5
    max_lo = jnp.maximum(lo, _TEST)
    cr = (max_lo + hi) * 0.5
    dr = (hi - max_lo) * 0.5
    both = left & right
    lj = jnp.minimum(cl - dl, cr - dr)
    rj = jnp.maximum(cl + dl, cr + dr)
    cb = (lj + rj) * 0.5
    db = (rj - lj) * 0.5
    new_c0 = jnp.where(both, cb, jnp.where(left, cl, cr))
    new_d0 = jnp.where(both, db, jnp.where(left, dl, dr))
    col = jax.lax.broadcasted_iota(jnp.int32, c.shape, 1)
    is_t = col == _TARGET
    oc_ref[...] = jnp.where(is_t, new_c0, c)
    od_ref[...] = jnp.where(is_t, new_d0, d)


def kernel(c, delta, idx):
    del idx
    spec = pl.BlockSpec((_BLOCK_ROWS, _COLS), lambda i: (i, 0))
    out_c, out_d = pl.pallas_call(
        _ifelse_kernel,
        grid=(_ROWS // _BLOCK_ROWS,),
        in_specs=[spec, spec],
        out_specs=[spec, spec],
        out_shape=[
            jax.ShapeDtypeStruct((_ROWS, _COLS), jnp.float32),
            jax.ShapeDtypeStruct((_ROWS, _COLS), jnp.float32),
        ],
    )(c, delta)
    return out_c, out_d


# speedup vs baseline: 8.7880x; 1.0279x over previous
import jax
import jax.numpy as jnp
from jax.experimental import pallas as pl

_TARGET = 0
_TEST = 0.0

_ROWS = 32768
_COLS = 256
_BLOCK_ROWS = 4096


def _ifelse_kernel(c_ref, d_ref, oc_ref, od_ref):
    c = c_ref[...]
    d = d_ref[...]
    c0 = c[:, _TARGET:_TARGET + 1]
    d0 = d[:, _TARGET:_TARGET + 1]
    lo = c0 - d0
    hi = c0 + d0
    left = lo <= _TEST
    right = hi > _TEST
    min_hi = jnp.minimum(hi, _TEST)
    cl = (lo + min_hi) * 0.5
    dl = (min_hi - lo) * 0.5
    max_lo = jnp.maximum(lo, _TEST)
    cr = (max_lo + hi) * 0.5
    dr = (hi - max_lo) * 0.5
    both = left & right
    lj = jnp.minimum(cl - dl, cr - dr)
    rj = jnp.maximum(cl + dl, cr + dr)
    cb = (lj + rj) * 0.5
    db = (rj - lj) * 0.5
    new_c0 = jnp.where(both, cb, jnp.where(left, cl, cr))
    new_d0 = jnp.where(both, db, jnp.where(left, dl, dr))
    col = jax.lax.broadcasted_iota(jnp.int32, c.shape, 1)
    is_t = col == _TARGET
    oc_ref[...] = jnp.where(is_t, new_c0, c)
    od_ref[...] = jnp.where(is_t, new_d0, d)


def kernel(c, delta, idx):
    del idx
    spec = pl.BlockSpec((_BLOCK_ROWS, _COLS), lambda i: (i, 0))
    out_c, out_d = pl.pallas_call(
        _ifelse_kernel,
        grid=(_ROWS // _BLOCK_ROWS,),
        in_specs=[spec, spec],
        out_specs=[spec, spec],
        out_shape=[
            jax.ShapeDtypeStruct((_ROWS, _COLS), jnp.float32),
            jax.ShapeDtypeStruct((_ROWS, _COLS), jnp.float32),
        ],
    )(c, delta)
    return out_c, out_d
